# Initial kernel scaffold; baseline (speedup 1.0000x reference)
import jax
import jax.numpy as jnp
from jax.experimental import pallas as pl


def kernel(node_feats, edge_attrs, senders, receivers, W1, W2, W3):
    raise NotImplementedError("write your pallas kernel here")


# speedup vs baseline: 3.8907x; 3.8907x over previous
import functools

import jax
import jax.numpy as jnp
import numpy as np
from jax import lax
from jax.experimental import pallas as pl
from jax.experimental.pallas import tpu as pltpu
from jax.experimental.pallas import tpu_sc as plsc

_N = 10000
_E = 320000
_MUL = 32
_AVG_NUM_NEIGHBORS = 32.0

_NC = 2
_NS = 16
_NW = _NC * _NS

_CHUNK = 80
_EPW = _E // _NW
_EPT = _E // _NS
_RPT = 640
_ACC_ROWS = _RPT * _NS
_LAST_ROWS = _N - _RPT * (_NS - 1)


def _perm128() -> np.ndarray:
    p = np.zeros((128, 128), np.float32)
    for c in range(_MUL):
        p[c, c] = 1.0
        for i in range(3):
            p[_MUL + 3 * c + i, _MUL + 32 * i + c] = 1.0
    return p


def _perm256() -> np.ndarray:
    p = np.zeros((256, 256), np.float32)
    for j in range(64):
        p[j, j] = 1.0
    for i in range(3):
        for c in range(64):
            p[64 + 64 * i + c, 64 + 3 * c + i] = 1.0
    return p


_P128 = _perm128()
_P256 = _perm256()


def _permute_body(x_ref, p_ref, o_ref):
    o_ref[...] = jnp.dot(x_ref[...], p_ref[...],
                         preferred_element_type=jnp.float32)


def _planarize_nodes(node_feats, p128):
    blk = 1000
    return pl.pallas_call(
        _permute_body,
        grid=(_N // blk,),
        in_specs=[pl.BlockSpec((blk, 128), lambda i: (i, 0)),
                  pl.BlockSpec((128, 128), lambda i: (0, 0))],
        out_specs=pl.BlockSpec((blk, 128), lambda i: (i, 0)),
        out_shape=jax.ShapeDtypeStruct((_N, 128), jnp.float32),
    )(node_feats, p128)


def _unpermute_body(x_ref, p_ref, o_ref):
    x = jnp.concatenate([x_ref[0], x_ref[1]], axis=1)
    o_ref[...] = jnp.dot(x, p_ref[...], preferred_element_type=jnp.float32)


def _unpermute_out(out_planar, p256):
    blk = 1000
    return pl.pallas_call(
        _unpermute_body,
        grid=(_N // blk,),
        in_specs=[pl.BlockSpec((2, blk, 128), lambda i: (0, i, 0)),
                  pl.BlockSpec((256, 256), lambda i: (0, 0))],
        out_specs=pl.BlockSpec((blk, 256), lambda i: (i, 0)),
        out_shape=jax.ShapeDtypeStruct((_N, 256), jnp.float32),
    )(out_planar, p256)


def _msg_body(g_ref, ea_ref, w1_ref, w2_ref, w3_ref, o_ref):
    g = g_ref[...]
    ea = ea_ref[...]
    y0 = ea[:, 0:1]
    h = jax.nn.swish(y0 * w1_ref[...])
    h = jax.nn.swish(jnp.dot(h, w2_ref[...], preferred_element_type=jnp.float32))
    mix = jnp.dot(h, w3_ref[...], preferred_element_type=jnp.float32)

    ms = g[:, 0:32]
    vx = g[:, 32:64]
    vy = g[:, 64:96]
    vz = g[:, 96:128]
    b1 = ea[:, 1:2]
    b2 = ea[:, 2:3]
    b3 = ea[:, 3:4]
    tp_s = (vx * b1 + vy * b2 + vz * b3) * np.float32(1.0 / np.sqrt(3.0))

    mix_a = mix[:, 0:32]
    mix_b = mix[:, 32:64]
    mv0 = mix[:, 64:96]
    mv1 = mix[:, 96:128]
    msv1 = ms * mv1
    lo = jnp.concatenate([ms * mix_a, tp_s * mix_b, vx * mv0, msv1 * b1], axis=1)
    hi = jnp.concatenate([vy * mv0, msv1 * b2, vz * mv0, msv1 * b3], axis=1)
    o_ref[0] = lo
    o_ref[1] = hi


def _messages(g, edge_attrs, w1, w2s, w3s):
    blk = 512
    return pl.pallas_call(
        _msg_body,
        grid=(_E // blk,),
        in_specs=[pl.BlockSpec((blk, 128), lambda i: (i, 0)),
                  pl.BlockSpec((blk, 4), lambda i: (i, 0)),
                  pl.BlockSpec((1, 64), lambda i: (0, 0)),
                  pl.BlockSpec((64, 64), lambda i: (0, 0)),
                  pl.BlockSpec((64, 128), lambda i: (0, 0))],
        out_specs=pl.BlockSpec((2, blk, 128), lambda i: (0, i, 0)),
        out_shape=jax.ShapeDtypeStruct((2, _E, 128), jnp.float32),
    )(g, edge_attrs, w1, w2s, w3s)


@functools.cache
def _build_sc_kernels():
    mesh = plsc.VectorSubcoreMesh(core_axis_name="c", subcore_axis_name="s",
                                  num_cores=_NC, num_subcores=_NS)

    @functools.partial(
        pl.kernel,
        out_type=jax.ShapeDtypeStruct((_E, 128), jnp.float32),
        mesh=mesh,
        scratch_types=[
            pltpu.VMEM((_CHUNK,), jnp.int32),
            pltpu.VMEM((_CHUNK, 128), jnp.float32),
            pltpu.SemaphoreType.DMA,
        ],
    )
    def gather_kernel(nf_hbm, senders_hbm, out_hbm, idx_v, rows_v, sem):
        wid = lax.axis_index("s") * _NC + lax.axis_index("c")
        base = wid * _EPW

        @pl.loop(0, _EPW // _CHUNK)
        def _(k):
            e0 = base + k * _CHUNK
            pltpu.sync_copy(senders_hbm.at[pl.ds(e0, _CHUNK)], idx_v)
            pltpu.async_copy(nf_hbm.at[idx_v], rows_v, sem).wait()
            pltpu.sync_copy(rows_v, out_hbm.at[pl.ds(e0, _CHUNK)])

    @functools.partial(
        pl.kernel,
        out_type=jax.ShapeDtypeStruct((2, _ACC_ROWS, 128), jnp.float32),
        mesh=mesh,
        scratch_types=[
            pltpu.VMEM((_CHUNK,), jnp.int32),
            pltpu.VMEM((_CHUNK, 128), jnp.float32),
            pltpu.VMEM_SHARED((_ACC_ROWS, 128), jnp.float32),
        ],
    )
    def scatter_kernel(m_hbm, recv_hbm, zeros_hbm, out_hbm, ridx_v, rows_v,
                       acc):
        cid = lax.axis_index("c")
        sid = lax.axis_index("s")

        r0 = sid * _RPT
        pltpu.sync_copy(zeros_hbm, rows_v)

        @pl.loop(0, _RPT // _CHUNK)
        def _(j):
            pltpu.sync_copy(rows_v, acc.at[pl.ds(r0 + j * _CHUNK, _CHUNK)])

        plsc.subcore_barrier()

        base = sid * _EPT

        @pl.loop(0, _EPT // _CHUNK)
        def _(k):
            e0 = base + k * _CHUNK
            pltpu.sync_copy(recv_hbm.at[pl.ds(e0, _CHUNK)], ridx_v)
            pltpu.sync_copy(m_hbm.at[cid].at[pl.ds(e0, _CHUNK)], rows_v)
            pltpu.sync_copy(rows_v, acc.at[ridx_v], add=True)

        plsc.subcore_barrier()

        @pl.loop(0, _RPT // _CHUNK)
        def _(j):
            pltpu.sync_copy(acc.at[pl.ds(r0 + j * _CHUNK, _CHUNK)], rows_v)
            pltpu.sync_copy(rows_v, out_hbm.at[cid].at[pl.ds(r0 + j * _CHUNK, _CHUNK)])

    return gather_kernel, scatter_kernel


def kernel(node_feats, edge_attrs, senders, receivers, W1, W2, W3):
    p128 = jnp.asarray(_P128)
    p256 = jnp.asarray(_P256)
    w2s = W2 * np.float32(1.0 / 8.0)
    w3s = W3 * np.float32(1.0 / (8.0 * np.sqrt(_AVG_NUM_NEIGHBORS)))
    zeros = jnp.zeros((_CHUNK, 128), jnp.float32)

    gather_kernel, scatter_kernel = _build_sc_kernels()
    nf_planar = _planarize_nodes(node_feats, p128)
    g = gather_kernel(nf_planar, senders)
    m = _messages(g, edge_attrs, W1, w2s, w3s)
    out_planar = scatter_kernel(m, receivers, zeros)[:, :_N, :]
    return _unpermute_out(out_planar, p256)
